# Initial kernel scaffold; baseline (speedup 1.0000x reference)
import jax
import jax.numpy as jnp
from jax.experimental import pallas as pl


def kernel(indices, table, W, b):
    raise NotImplementedError("write your pallas kernel here")


# speedup vs baseline: 7.6192x; 7.6192x over previous
import functools

import jax
import jax.numpy as jnp
from jax import lax
from jax.experimental import pallas as pl
from jax.experimental.pallas import tpu as pltpu
from jax.experimental.pallas import tpu_sc as plsc

B = 16384
F = 26
D = 64
NC, NS = 2, 16
NW = NC * NS
E_PER_W = B // NW
CH = 32
NCHUNK = E_PER_W // CH
GI = 104
G = CH * F // GI
IDX_ROWS_PER_W = E_PER_W * F // GI
LANES = 16
KD = D // LANES


def _sc_gather_sum(idx2d, table):
    mesh = plsc.VectorSubcoreMesh(core_axis_name="c", subcore_axis_name="s")

    @functools.partial(
        pl.kernel,
        out_type=jax.ShapeDtypeStruct((B, D), jnp.float32),
        mesh=mesh,
        scratch_types=[
            pltpu.VMEM((G, GI), jnp.int32),
            pltpu.VMEM((CH * F, D), jnp.float32),
            pltpu.VMEM((CH, D), jnp.float32),
            pltpu.SemaphoreType.DMA,
        ],
        compiler_params=pltpu.CompilerParams(use_tc_tiling_on_sc=False),
    )
    def k(idx_hbm, table_hbm, out_hbm, idx_v, rows_v, out_v, sem):
        wid = lax.axis_index("s") * NC + lax.axis_index("c")
        idx_row_base = wid * IDX_ROWS_PER_W
        out_base = wid * E_PER_W

        def chunk_body(c, _):
            pltpu.sync_copy(idx_hbm.at[pl.ds(idx_row_base + c * G, G)], idx_v)
            copies = []
            for j in range(G):
                copies.append(pltpu.async_copy(
                    table_hbm.at[idx_v.at[j]],
                    rows_v.at[pl.ds(j * GI, GI)],
                    sem,
                ))
            for cp in copies:
                cp.wait()

            def ent_body(e, _):
                r0 = e * F
                for kk in range(KD):
                    acc = rows_v[r0, pl.ds(kk * LANES, LANES)]
                    for f in range(1, F):
                        acc = acc + rows_v[r0 + f, pl.ds(kk * LANES, LANES)]
                    out_v[e, pl.ds(kk * LANES, LANES)] = acc
                return 0

            lax.fori_loop(0, CH, ent_body, 0)
            pltpu.sync_copy(out_v, out_hbm.at[pl.ds(out_base + c * CH, CH)])
            return 0

        lax.fori_loop(0, NCHUNK, chunk_body, 0)

    return k(idx2d, table)


def _tc_mlp(summed, W, b):
    BM = 2048

    def body(x_ref, w_ref, b_ref, o_ref):
        y = jnp.dot(x_ref[...], w_ref[...], preferred_element_type=jnp.float32)
        o_ref[...] = jnp.maximum(y + b_ref[...], 0.0)

    return pl.pallas_call(
        body,
        grid=(B // BM,),
        in_specs=[
            pl.BlockSpec((BM, D), lambda i: (i, 0)),
            pl.BlockSpec((D, D), lambda i: (0, 0)),
            pl.BlockSpec((1, D), lambda i: (0, 0)),
        ],
        out_specs=pl.BlockSpec((BM, D), lambda i: (i, 0)),
        out_shape=jax.ShapeDtypeStruct((B, D), jnp.float32),
    )(summed, W, b.reshape(1, D))


def kernel(indices, table, W, b):
    idx2d = indices.reshape(B * F // GI, GI)
    summed = _sc_gather_sum(idx2d, table)
    return _tc_mlp(summed, W, b)


# speedup vs baseline: 9.1266x; 1.1978x over previous
import functools

import jax
import jax.numpy as jnp
from jax import lax
from jax.experimental import pallas as pl
from jax.experimental.pallas import tpu as pltpu
from jax.experimental.pallas import tpu_sc as plsc

B = 16384
F = 26
D = 64
NC, NS = 2, 16
NW = NC * NS
E_PER_W = B // NW
CH = 32
NCHUNK = E_PER_W // CH
GI = 104
G = CH * F // GI
IDX_ROWS_PER_W = E_PER_W * F // GI
LANES = 16
KD = D // LANES


def _sc_gather_sum(idx2d, table):
    mesh = plsc.VectorSubcoreMesh(core_axis_name="c", subcore_axis_name="s")

    @functools.partial(
        pl.kernel,
        out_type=jax.ShapeDtypeStruct((B, D), jnp.float32),
        mesh=mesh,
        scratch_types=[
            pltpu.VMEM((2, G, GI), jnp.int32),
            pltpu.VMEM((2, CH * F, D), jnp.float32),
            pltpu.VMEM((2, CH, D), jnp.float32),
            pltpu.SemaphoreType.DMA,
            pltpu.SemaphoreType.DMA,
        ],
        compiler_params=pltpu.CompilerParams(use_tc_tiling_on_sc=False),
    )
    def k(idx_hbm, table_hbm, out_hbm, idx_v, rows_v, out_v, sem0, sem1):
        wid = lax.axis_index("s") * NC + lax.axis_index("c")
        idx_row_base = wid * IDX_ROWS_PER_W
        out_base = wid * E_PER_W
        sems = (sem0, sem1)

        def issue(c, bslot):
            pltpu.sync_copy(idx_hbm.at[pl.ds(idx_row_base + c * G, G)],
                            idx_v.at[bslot])
            for j in range(G):
                pltpu.async_copy(
                    table_hbm.at[idx_v.at[bslot, j]],
                    rows_v.at[bslot, pl.ds(j * GI, GI)],
                    sems[bslot],
                )

        def drain(bslot):
            for j in range(G):
                pltpu.make_async_copy(
                    table_hbm.at[idx_v.at[bslot, j]],
                    rows_v.at[bslot, pl.ds(j * GI, GI)],
                    sems[bslot],
                ).wait()

        def accumulate(c, bslot):
            def ent_body(e, _):
                r0 = e * F
                for kk in range(KD):
                    acc = rows_v[bslot, r0, pl.ds(kk * LANES, LANES)]
                    for f in range(1, F):
                        acc = acc + rows_v[bslot, r0 + f, pl.ds(kk * LANES, LANES)]
                    out_v[bslot, e, pl.ds(kk * LANES, LANES)] = acc
                return 0

            lax.fori_loop(0, CH, ent_body, 0)
            pltpu.sync_copy(out_v.at[bslot],
                            out_hbm.at[pl.ds(out_base + c * CH, CH)])

        issue(0, 0)
        issue(1, 1)

        @pl.loop(0, NCHUNK, step=2)
        def chunk_body(g):
            for bslot in range(2):
                c = g + bslot
                drain(bslot)
                accumulate(c, bslot)

                @pl.when(c + 2 < NCHUNK)
                def _():
                    issue(c + 2, bslot)

    return k(idx2d, table)


def _tc_mlp(summed, W, b):
    BM = 2048

    def body(x_ref, w_ref, b_ref, o_ref):
        y = jnp.dot(x_ref[...], w_ref[...], preferred_element_type=jnp.float32)
        o_ref[...] = jnp.maximum(y + b_ref[...], 0.0)

    return pl.pallas_call(
        body,
        grid=(B // BM,),
        in_specs=[
            pl.BlockSpec((BM, D), lambda i: (i, 0)),
            pl.BlockSpec((D, D), lambda i: (0, 0)),
            pl.BlockSpec((1, D), lambda i: (0, 0)),
        ],
        out_specs=pl.BlockSpec((BM, D), lambda i: (i, 0)),
        out_shape=jax.ShapeDtypeStruct((B, D), jnp.float32),
    )(summed, W, b.reshape(1, D))


def kernel(indices, table, W, b):
    idx2d = indices.reshape(B * F // GI, GI)
    summed = _sc_gather_sum(idx2d, table)
    return _tc_mlp(summed, W, b)
